# Initial kernel scaffold; baseline (speedup 1.0000x reference)
import jax
import jax.numpy as jnp
from jax.experimental import pallas as pl


def kernel(node_attrs, node_feats, edge_attrs, edge_feats, edge_index, W_src, W_tgt, W_up, W_res, W_skip, W_tp1, W_tp2, W_tp3, W_tp4, W_d1, W_d2, W_1, W_2, alpha, beta):
    raise NotImplementedError("write your pallas kernel here")


# speedup vs baseline: 2.6222x; 2.6222x over previous
import math

import jax
import jax.numpy as jnp
from jax import lax
from jax.experimental import pallas as pl
from jax.experimental.pallas import tpu as pltpu
from jax.experimental.pallas import tpu_sc as plsc

F32 = jnp.float32

_N = 10000
_E = 320000
_DA = 10
_DF = 128
_DEF = 8
_NP = 10240
_NW = 32
_EPW = _E // _NW
_CH = 80
_NCH = _EPW // _CH
_RPT = _NP // 16


def _node_tables_body(na_ref, nf_ref, wsf_ref, wdf_ref, wup_ref, wres_ref,
                      wskip_ref, tsu_ref, td_ref, res_ref, sc_ref):
    na = na_ref[...]
    nf = nf_ref[...]
    tsu_ref[:, 0:_DF] = jnp.dot(na, wsf_ref[...], preferred_element_type=F32)
    tsu_ref[:, _DF:2 * _DF] = jnp.dot(nf, wup_ref[...],
                                      preferred_element_type=F32)
    td_ref[...] = jnp.dot(na, wdf_ref[...], preferred_element_type=F32)
    res_ref[...] = jnp.dot(nf, wres_ref[...], preferred_element_type=F32)
    sc_ref[...] = jnp.dot(nf, wskip_ref[...], preferred_element_type=F32)


def _node_tables(na, nf, wsf, wdf, wupn, wresn, wskipn):
    tn = 1000
    grid = (_N // tn,)
    full = lambda shape: pl.BlockSpec(shape, lambda i: (0, 0))
    return pl.pallas_call(
        _node_tables_body,
        grid=grid,
        in_specs=[
            pl.BlockSpec((tn, _DA), lambda i: (i, 0)),
            pl.BlockSpec((tn, _DF), lambda i: (i, 0)),
            full((_DA, _DF)), full((_DA, _DF)),
            full((_DF, _DF)), full((_DF, _DF)), full((_DF, _DF)),
        ],
        out_specs=[
            pl.BlockSpec((tn, 2 * _DF), lambda i: (i, 0)),
            pl.BlockSpec((tn, _DF), lambda i: (i, 0)),
            pl.BlockSpec((tn, _DF), lambda i: (i, 0)),
            pl.BlockSpec((tn, _DF), lambda i: (i, 0)),
        ],
        out_shape=[
            jax.ShapeDtypeStruct((_N, 2 * _DF), F32),
            jax.ShapeDtypeStruct((_N, _DF), F32),
            jax.ShapeDtypeStruct((_N, _DF), F32),
            jax.ShapeDtypeStruct((_N, _DF), F32),
        ],
        compiler_params=pltpu.CompilerParams(
            dimension_semantics=("parallel",)),
    )(na, nf, wsf, wdf, wupn, wresn, wskipn)


def _sc_gather_body(tsu_hbm, td_hbm, src_hbm, dst_hbm, ga_hbm, gb_hbm,
                    sidx, didx, bufs, bufd, sem):
    cid = lax.axis_index("c")
    sid = lax.axis_index("s")
    base = (sid * 2 + cid) * _EPW

    def chunk(j, carry):
        b = base + j * _CH
        pltpu.sync_copy(src_hbm.at[pl.ds(b, _CH)], sidx)
        pltpu.sync_copy(dst_hbm.at[pl.ds(b, _CH)], didx)
        pltpu.async_copy(tsu_hbm.at[sidx], bufs, sem).wait()
        pltpu.async_copy(td_hbm.at[didx], bufd, sem).wait()
        pltpu.sync_copy(bufs, ga_hbm.at[pl.ds(b, _CH)])
        pltpu.sync_copy(bufd, gb_hbm.at[pl.ds(b, _CH)])
        return carry

    lax.fori_loop(0, _NCH, chunk, 0)


def _gather_phase(tsu, td, src, dst):
    sc_gather = pl.kernel(
        _sc_gather_body,
        out_type=(
            jax.ShapeDtypeStruct((_E, 2 * _DF), F32),
            jax.ShapeDtypeStruct((_E, _DF), F32),
        ),
        mesh=plsc.VectorSubcoreMesh(core_axis_name="c", subcore_axis_name="s",
                                    num_cores=2, num_subcores=16),
        scratch_types=[
            pltpu.VMEM((_CH,), jnp.int32),
            pltpu.VMEM((_CH,), jnp.int32),
            pltpu.VMEM((_CH, 2 * _DF), F32),
            pltpu.VMEM((_CH, _DF), F32),
            pltpu.SemaphoreType.DMA,
        ],
    )
    return sc_gather(tsu, td, src, dst)


def _edge_mlp_body(ga_ref, gb_ref, ef_ref, ea_ref, w1e_ref, wde_ref, w2_ref,
                   w3_ref, w4_ref, wd2_ref, mji_ref, dens_ref):
    ga = ga_ref[...]
    gb = gb_ref[...]
    ef = ef_ref[...]
    h1 = ga[:, 0:64] + gb[:, 0:64] + jnp.dot(ef, w1e_ref[...],
                                             preferred_element_type=F32)
    h1 = h1 * jax.nn.sigmoid(h1)
    d1 = ga[:, 64:128] + gb[:, 64:128] + jnp.dot(ef, wde_ref[...],
                                                 preferred_element_type=F32)
    d1 = d1 * jax.nn.sigmoid(d1)
    h2 = jnp.dot(h1, w2_ref[...], preferred_element_type=F32)
    h2 = h2 * jax.nn.sigmoid(h2)
    h3 = jnp.dot(h2, w3_ref[...], preferred_element_type=F32)
    h3 = h3 * jax.nn.sigmoid(h3)
    tpw = jnp.dot(h3, w4_ref[...], preferred_element_type=F32)
    mji_ref[...] = ga[:, _DF:2 * _DF] * (ea_ref[...] * tpw)
    dd = jnp.dot(d1, wd2_ref[...], preferred_element_type=F32)
    dens_ref[...] = jnp.tanh(dd * dd)


def _edge_mlp(ga, gb, ef, ea, w1e, wde, w2n, w3n, w4n, wd2n):
    te = 2000
    grid = (_E // te,)
    full = lambda shape: pl.BlockSpec(shape, lambda i: (0, 0))
    return pl.pallas_call(
        _edge_mlp_body,
        grid=grid,
        in_specs=[
            pl.BlockSpec((te, 2 * _DF), lambda i: (i, 0)),
            pl.BlockSpec((te, _DF), lambda i: (i, 0)),
            pl.BlockSpec((te, _DEF), lambda i: (i, 0)),
            pl.BlockSpec((te, 1), lambda i: (i, 0)),
            full((_DEF, 64)), full((_DEF, 64)),
            full((64, 64)), full((64, 64)), full((64, _DF)), full((64, 1)),
        ],
        out_specs=[
            pl.BlockSpec((te, _DF), lambda i: (i, 0)),
            pl.BlockSpec((te, 1), lambda i: (i, 0)),
        ],
        out_shape=[
            jax.ShapeDtypeStruct((_E, _DF), F32),
            jax.ShapeDtypeStruct((_E, 1), F32),
        ],
        compiler_params=pltpu.CompilerParams(
            dimension_semantics=("parallel",)),
    )(ga, gb, ef, ea, w1e, wde, w2n, w3n, w4n, wd2n)


def _sc_scatter_body(dst_hbm, mji_hbm, de_hbm, zm_hbm, zd_hbm,
                     msgp_hbm, denp_hbm,
                     didx, bufm, bufe, sem, acc_m, acc_d):
    cid = lax.axis_index("c")
    sid = lax.axis_index("s")
    r0 = sid * _RPT
    pltpu.sync_copy(zm_hbm.at[pl.ds(r0, _RPT)], acc_m.at[pl.ds(r0, _RPT)])
    pltpu.sync_copy(zd_hbm.at[pl.ds(r0, _RPT)], acc_d.at[pl.ds(r0, _RPT)])
    plsc.subcore_barrier()
    base = (sid * 2 + cid) * _EPW

    def chunk(j, carry):
        b = base + j * _CH
        pltpu.sync_copy(dst_hbm.at[pl.ds(b, _CH)], didx)
        pltpu.sync_copy(mji_hbm.at[pl.ds(b, _CH)], bufm)
        pltpu.sync_copy(de_hbm.at[pl.ds(b, _CH)], bufe)
        pltpu.sync_copy(bufm, acc_m.at[didx], add=True)
        pltpu.sync_copy(bufe, acc_d.at[didx], add=True)
        return carry

    lax.fori_loop(0, _NCH, chunk, 0)
    plsc.subcore_barrier()
    pltpu.sync_copy(acc_m.at[pl.ds(r0, _RPT)],
                    msgp_hbm.at[cid, pl.ds(r0, _RPT)])
    pltpu.sync_copy(acc_d.at[pl.ds(r0, _RPT)],
                    denp_hbm.at[cid, pl.ds(r0, _RPT)])


def _scatter_phase(dst, mji, de):
    sc_scatter = pl.kernel(
        _sc_scatter_body,
        out_type=(
            jax.ShapeDtypeStruct((2, _NP, _DF), F32),
            jax.ShapeDtypeStruct((2, _NP), F32),
        ),
        mesh=plsc.VectorSubcoreMesh(core_axis_name="c", subcore_axis_name="s",
                                    num_cores=2, num_subcores=16),
        scratch_types=[
            pltpu.VMEM((_CH,), jnp.int32),
            pltpu.VMEM((_CH, _DF), F32),
            pltpu.VMEM((_CH,), F32),
            pltpu.SemaphoreType.DMA,
            pltpu.VMEM_SHARED((_NP, _DF), F32),
            pltpu.VMEM_SHARED((_NP,), F32),
        ],
    )
    zm = jnp.zeros((_NP, _DF), F32)
    zd = jnp.zeros((_NP,), F32)
    return sc_scatter(dst, mji, de, zm, zd)


def _final_body(msgp_ref, denp_ref, res_ref, w1_ref, w2_ref, a_ref, b_ref,
                out_ref):
    msg = msgp_ref[0] + msgp_ref[1]
    den = denp_ref[0] + denp_ref[1]
    lin = jnp.dot(msg, w1_ref[...], preferred_element_type=F32)
    m = lin / (den * b_ref[0, 0] + a_ref[0, 0]) + res_ref[...]
    m = m * jax.nn.sigmoid(m)
    out_ref[...] = jnp.dot(m, w2_ref[...], preferred_element_type=F32)


def _final(msgp, denp3, resp, w1n, w2n, a2, b2):
    tn = 1024
    grid = (_NP // tn,)
    full = lambda shape: pl.BlockSpec(shape, lambda i: (0, 0))
    smem = pl.BlockSpec((1, 1), lambda i: (0, 0), memory_space=pltpu.SMEM)
    return pl.pallas_call(
        _final_body,
        grid=grid,
        in_specs=[
            pl.BlockSpec((2, tn, _DF), lambda i: (0, i, 0)),
            pl.BlockSpec((2, tn, 1), lambda i: (0, i, 0)),
            pl.BlockSpec((tn, _DF), lambda i: (i, 0)),
            full((_DF, _DF)), full((_DF, _DF)),
            smem, smem,
        ],
        out_specs=pl.BlockSpec((tn, _DF), lambda i: (i, 0)),
        out_shape=jax.ShapeDtypeStruct((_NP, _DF), F32),
        compiler_params=pltpu.CompilerParams(
            dimension_semantics=("parallel",)),
    )(msgp, denp3, resp, w1n, w2n, a2, b2)


def kernel(node_attrs, node_feats, edge_attrs, edge_feats, edge_index,
           W_src, W_tgt, W_up, W_res, W_skip,
           W_tp1, W_tp2, W_tp3, W_tp4, W_d1, W_d2, W_1, W_2, alpha, beta):
    s_attr = math.sqrt(W_src.shape[0])
    s_aug = math.sqrt(W_tp1.shape[0])
    s_mid = math.sqrt(W_tp2.shape[0])
    s_feat = math.sqrt(W_up.shape[0])

    src = edge_index[:, 0].astype(jnp.int32)
    dst = edge_index[:, 1].astype(jnp.int32)

    cfold = 1.0 / (s_attr * s_aug)
    lo, hi = _DEF, _DEF + _DF
    wsf = jnp.concatenate([W_src @ W_tp1[lo:hi], W_src @ W_d1[lo:hi]],
                          axis=1) * cfold
    wdf = jnp.concatenate([W_tgt @ W_tp1[hi:], W_tgt @ W_d1[hi:]],
                          axis=1) * cfold

    tsu, td, resv, scv = _node_tables(
        node_attrs, node_feats, wsf, wdf,
        W_up / s_feat, W_res / s_feat, W_skip / s_feat)

    ga, gb = _gather_phase(tsu, td, src, dst)

    mji, dens_e = _edge_mlp(
        ga, gb, edge_feats, edge_attrs,
        W_tp1[0:_DEF] / s_aug, W_d1[0:_DEF] / s_aug,
        W_tp2 / s_mid, W_tp3 / s_mid, W_tp4 / s_mid, W_d2 / s_mid)

    msgp, denp = _scatter_phase(dst, mji, dens_e.reshape(_E))

    resp = jnp.pad(resv, ((0, _NP - _N), (0, 0)))
    out_m = _final(msgp, denp.reshape(2, _NP, 1), resp,
                   W_1 / s_feat, W_2 / s_feat,
                   alpha.reshape(1, 1), beta.reshape(1, 1))

    return (out_m[:_N].reshape(_N, _DF, 1), scv)


# speedup vs baseline: 3.5558x; 1.3560x over previous
import math

import jax
import jax.numpy as jnp
from jax import lax
from jax.experimental import pallas as pl
from jax.experimental.pallas import tpu as pltpu
from jax.experimental.pallas import tpu_sc as plsc

F32 = jnp.float32

_N = 10000
_E = 320000
_DA = 10
_DF = 128
_DEF = 8
_NP = 10240
_NW = 32
_EPW = _E // _NW
_CH = 80
_NCH = _EPW // _CH
_RPT = _NP // 16


def _node_tables_body(na_ref, nf_ref, wsf_ref, wdf_ref, wup_ref, wres_ref,
                      wskip_ref, tsu_ref, td_ref, res_ref, sc_ref):
    na = na_ref[...]
    nf = nf_ref[...]
    tsu_ref[:, 0:_DF] = jnp.dot(na, wsf_ref[...], preferred_element_type=F32)
    tsu_ref[:, _DF:2 * _DF] = jnp.dot(nf, wup_ref[...],
                                      preferred_element_type=F32)
    td_ref[...] = jnp.dot(na, wdf_ref[...], preferred_element_type=F32)
    res_ref[...] = jnp.dot(nf, wres_ref[...], preferred_element_type=F32)
    sc_ref[...] = jnp.dot(nf, wskip_ref[...], preferred_element_type=F32)


def _node_tables(na, nf, wsf, wdf, wupn, wresn, wskipn):
    tn = 1000
    grid = (_N // tn,)
    full = lambda shape: pl.BlockSpec(shape, lambda i: (0, 0))
    return pl.pallas_call(
        _node_tables_body,
        grid=grid,
        in_specs=[
            pl.BlockSpec((tn, _DA), lambda i: (i, 0)),
            pl.BlockSpec((tn, _DF), lambda i: (i, 0)),
            full((_DA, _DF)), full((_DA, _DF)),
            full((_DF, _DF)), full((_DF, _DF)), full((_DF, _DF)),
        ],
        out_specs=[
            pl.BlockSpec((tn, 2 * _DF), lambda i: (i, 0)),
            pl.BlockSpec((tn, _DF), lambda i: (i, 0)),
            pl.BlockSpec((tn, _DF), lambda i: (i, 0)),
            pl.BlockSpec((tn, _DF), lambda i: (i, 0)),
        ],
        out_shape=[
            jax.ShapeDtypeStruct((_N, 2 * _DF), F32),
            jax.ShapeDtypeStruct((_N, _DF), F32),
            jax.ShapeDtypeStruct((_N, _DF), F32),
            jax.ShapeDtypeStruct((_N, _DF), F32),
        ],
        compiler_params=pltpu.CompilerParams(
            dimension_semantics=("parallel",)),
    )(na, nf, wsf, wdf, wupn, wresn, wskipn)


def _sc_gather_body(tsu_hbm, td_hbm, srcr_hbm, dstr_hbm, ga_hbm, gb_hbm,
                    sidx, didx, bufs, bufd, sem):
    cid = lax.axis_index("c")
    sid = lax.axis_index("s")
    wid = sid * 2 + cid
    base = wid * _EPW
    pltpu.sync_copy(srcr_hbm.at[wid], sidx)
    pltpu.sync_copy(dstr_hbm.at[wid], didx)

    def start_gather(j, p):
        pltpu.async_copy(tsu_hbm.at[sidx.at[j]], bufs.at[p], sem.at[p])
        pltpu.async_copy(td_hbm.at[didx.at[j]], bufd.at[p], sem.at[p])

    start_gather(0, 0)

    def chunk(j, carry):
        p = lax.rem(j, 2)

        @pl.when(j + 1 < _NCH)
        def _():
            start_gather(j + 1, 1 - p)

        pltpu.make_async_copy(tsu_hbm.at[sidx.at[j]], bufs.at[p],
                              sem.at[p]).wait()
        pltpu.make_async_copy(td_hbm.at[didx.at[j]], bufd.at[p],
                              sem.at[p]).wait()
        b = base + j * _CH
        pltpu.sync_copy(bufs.at[p], ga_hbm.at[pl.ds(b, _CH)])
        pltpu.sync_copy(bufd.at[p], gb_hbm.at[pl.ds(b, _CH)])
        return carry

    lax.fori_loop(0, _NCH, chunk, 0)


def _gather_phase(tsu, td, src, dst):
    sc_gather = pl.kernel(
        _sc_gather_body,
        out_type=(
            jax.ShapeDtypeStruct((_E, 2 * _DF), F32),
            jax.ShapeDtypeStruct((_E, _DF), F32),
        ),
        mesh=plsc.VectorSubcoreMesh(core_axis_name="c", subcore_axis_name="s",
                                    num_cores=2, num_subcores=16),
        scratch_types=[
            pltpu.VMEM((_NCH, _CH), jnp.int32),
            pltpu.VMEM((_NCH, _CH), jnp.int32),
            pltpu.VMEM((2, _CH, 2 * _DF), F32),
            pltpu.VMEM((2, _CH, _DF), F32),
            pltpu.SemaphoreType.DMA((2,)),
        ],
    )
    return sc_gather(tsu, td, src.reshape(_NW, _NCH, _CH),
                     dst.reshape(_NW, _NCH, _CH))


def _edge_mlp_body(ga_ref, gb_ref, ef_ref, ea_ref, w1e_ref, wde_ref, w2_ref,
                   w3_ref, w4_ref, wd2_ref, mji_ref, dens_ref):
    ga = ga_ref[...]
    gb = gb_ref[...]
    ef = ef_ref[...]
    h1 = ga[:, 0:64] + gb[:, 0:64] + jnp.dot(ef, w1e_ref[...],
                                             preferred_element_type=F32)
    h1 = h1 * jax.nn.sigmoid(h1)
    d1 = ga[:, 64:128] + gb[:, 64:128] + jnp.dot(ef, wde_ref[...],
                                                 preferred_element_type=F32)
    d1 = d1 * jax.nn.sigmoid(d1)
    h2 = jnp.dot(h1, w2_ref[...], preferred_element_type=F32)
    h2 = h2 * jax.nn.sigmoid(h2)
    h3 = jnp.dot(h2, w3_ref[...], preferred_element_type=F32)
    h3 = h3 * jax.nn.sigmoid(h3)
    tpw = jnp.dot(h3, w4_ref[...], preferred_element_type=F32)
    mji_ref[...] = ga[:, _DF:2 * _DF] * (ea_ref[...] * tpw)
    dd = jnp.dot(d1, wd2_ref[...], preferred_element_type=F32)
    dens_ref[...] = jnp.tanh(dd * dd)


def _edge_mlp(ga, gb, ef, ea, w1e, wde, w2n, w3n, w4n, wd2n):
    te = 2000
    grid = (_E // te,)
    full = lambda shape: pl.BlockSpec(shape, lambda i: (0, 0))
    return pl.pallas_call(
        _edge_mlp_body,
        grid=grid,
        in_specs=[
            pl.BlockSpec((te, 2 * _DF), lambda i: (i, 0)),
            pl.BlockSpec((te, _DF), lambda i: (i, 0)),
            pl.BlockSpec((te, _DEF), lambda i: (i, 0)),
            pl.BlockSpec((te, 1), lambda i: (i, 0)),
            full((_DEF, 64)), full((_DEF, 64)),
            full((64, 64)), full((64, 64)), full((64, _DF)), full((64, 1)),
        ],
        out_specs=[
            pl.BlockSpec((te, _DF), lambda i: (i, 0)),
            pl.BlockSpec((te, 1), lambda i: (i, 0)),
        ],
        out_shape=[
            jax.ShapeDtypeStruct((_E, _DF), F32),
            jax.ShapeDtypeStruct((_E, 1), F32),
        ],
        compiler_params=pltpu.CompilerParams(
            dimension_semantics=("parallel",)),
    )(ga, gb, ef, ea, w1e, wde, w2n, w3n, w4n, wd2n)


def _sc_scatter_body(dstr_hbm, mji_hbm, de_hbm, zm_hbm, zd_hbm,
                     msgp_hbm, denp_hbm,
                     didx, bufm, bufe, sem, acc_m, acc_d):
    cid = lax.axis_index("c")
    sid = lax.axis_index("s")
    wid = sid * 2 + cid
    r0 = sid * _RPT
    pltpu.sync_copy(zm_hbm.at[pl.ds(r0, _RPT)], acc_m.at[pl.ds(r0, _RPT)])
    pltpu.sync_copy(zd_hbm.at[pl.ds(r0, _RPT)], acc_d.at[pl.ds(r0, _RPT)])
    pltpu.sync_copy(dstr_hbm.at[wid], didx)
    plsc.subcore_barrier()
    base = wid * _EPW

    def start_load(j, p):
        b = base + j * _CH
        pltpu.async_copy(mji_hbm.at[pl.ds(b, _CH)], bufm.at[p], sem.at[p])
        pltpu.async_copy(de_hbm.at[pl.ds(b, _CH)], bufe.at[p], sem.at[p])

    start_load(0, 0)

    def chunk(j, carry):
        p = lax.rem(j, 2)

        @pl.when(j + 1 < _NCH)
        def _():
            start_load(j + 1, 1 - p)

        b = base + j * _CH
        pltpu.make_async_copy(mji_hbm.at[pl.ds(b, _CH)], bufm.at[p],
                              sem.at[p]).wait()
        pltpu.make_async_copy(de_hbm.at[pl.ds(b, _CH)], bufe.at[p],
                              sem.at[p]).wait()
        pltpu.sync_copy(bufm.at[p], acc_m.at[didx.at[j]], add=True)
        pltpu.sync_copy(bufe.at[p], acc_d.at[didx.at[j]], add=True)
        return carry

    lax.fori_loop(0, _NCH, chunk, 0)
    plsc.subcore_barrier()
    pltpu.sync_copy(acc_m.at[pl.ds(r0, _RPT)],
                    msgp_hbm.at[cid, pl.ds(r0, _RPT)])
    pltpu.sync_copy(acc_d.at[pl.ds(r0, _RPT)],
                    denp_hbm.at[cid, pl.ds(r0, _RPT)])


def _scatter_phase(dst, mji, de):
    sc_scatter = pl.kernel(
        _sc_scatter_body,
        out_type=(
            jax.ShapeDtypeStruct((2, _NP, _DF), F32),
            jax.ShapeDtypeStruct((2, _NP), F32),
        ),
        mesh=plsc.VectorSubcoreMesh(core_axis_name="c", subcore_axis_name="s",
                                    num_cores=2, num_subcores=16),
        scratch_types=[
            pltpu.VMEM((_NCH, _CH), jnp.int32),
            pltpu.VMEM((2, _CH, _DF), F32),
            pltpu.VMEM((2, _CH), F32),
            pltpu.SemaphoreType.DMA((2,)),
            pltpu.VMEM_SHARED((_NP, _DF), F32),
            pltpu.VMEM_SHARED((_NP,), F32),
        ],
    )
    zm = jnp.zeros((_NP, _DF), F32)
    zd = jnp.zeros((_NP,), F32)
    return sc_scatter(dst.reshape(_NW, _NCH, _CH), mji, de, zm, zd)


def _final_body(msgp_ref, denp_ref, res_ref, w1_ref, w2_ref, a_ref, b_ref,
                out_ref):
    msg = msgp_ref[0] + msgp_ref[1]
    den = denp_ref[0] + denp_ref[1]
    lin = jnp.dot(msg, w1_ref[...], preferred_element_type=F32)
    m = lin / (den * b_ref[0, 0] + a_ref[0, 0]) + res_ref[...]
    m = m * jax.nn.sigmoid(m)
    out_ref[...] = jnp.dot(m, w2_ref[...], preferred_element_type=F32)


def _final(msgp, denp3, resp, w1n, w2n, a2, b2):
    tn = 1024
    grid = (_NP // tn,)
    full = lambda shape: pl.BlockSpec(shape, lambda i: (0, 0))
    smem = pl.BlockSpec((1, 1), lambda i: (0, 0), memory_space=pltpu.SMEM)
    return pl.pallas_call(
        _final_body,
        grid=grid,
        in_specs=[
            pl.BlockSpec((2, tn, _DF), lambda i: (0, i, 0)),
            pl.BlockSpec((2, tn, 1), lambda i: (0, i, 0)),
            pl.BlockSpec((tn, _DF), lambda i: (i, 0)),
            full((_DF, _DF)), full((_DF, _DF)),
            smem, smem,
        ],
        out_specs=pl.BlockSpec((tn, _DF), lambda i: (i, 0)),
        out_shape=jax.ShapeDtypeStruct((_NP, _DF), F32),
        compiler_params=pltpu.CompilerParams(
            dimension_semantics=("parallel",)),
    )(msgp, denp3, resp, w1n, w2n, a2, b2)


def kernel(node_attrs, node_feats, edge_attrs, edge_feats, edge_index,
           W_src, W_tgt, W_up, W_res, W_skip,
           W_tp1, W_tp2, W_tp3, W_tp4, W_d1, W_d2, W_1, W_2, alpha, beta):
    s_attr = math.sqrt(W_src.shape[0])
    s_aug = math.sqrt(W_tp1.shape[0])
    s_mid = math.sqrt(W_tp2.shape[0])
    s_feat = math.sqrt(W_up.shape[0])

    src = edge_index[:, 0].astype(jnp.int32)
    dst = edge_index[:, 1].astype(jnp.int32)

    cfold = 1.0 / (s_attr * s_aug)
    lo, hi = _DEF, _DEF + _DF
    wsf = jnp.concatenate([W_src @ W_tp1[lo:hi], W_src @ W_d1[lo:hi]],
                          axis=1) * cfold
    wdf = jnp.concatenate([W_tgt @ W_tp1[hi:], W_tgt @ W_d1[hi:]],
                          axis=1) * cfold

    tsu, td, resv, scv = _node_tables(
        node_attrs, node_feats, wsf, wdf,
        W_up / s_feat, W_res / s_feat, W_skip / s_feat)

    ga, gb = _gather_phase(tsu, td, src, dst)

    mji, dens_e = _edge_mlp(
        ga, gb, edge_feats, edge_attrs,
        W_tp1[0:_DEF] / s_aug, W_d1[0:_DEF] / s_aug,
        W_tp2 / s_mid, W_tp3 / s_mid, W_tp4 / s_mid, W_d2 / s_mid)

    msgp, denp = _scatter_phase(dst, mji, dens_e.reshape(_E))

    resp = jnp.pad(resv, ((0, _NP - _N), (0, 0)))
    out_m = _final(msgp, denp.reshape(2, _NP, 1), resp,
                   W_1 / s_feat, W_2 / s_feat,
                   alpha.reshape(1, 1), beta.reshape(1, 1))

    return (out_m[:_N].reshape(_N, _DF, 1), scv)


# speedup vs baseline: 4.0678x; 1.1440x over previous
import math

import jax
import jax.numpy as jnp
from jax import lax
from jax.experimental import pallas as pl
from jax.experimental.pallas import tpu as pltpu
from jax.experimental.pallas import tpu_sc as plsc

F32 = jnp.float32
BF16 = jnp.bfloat16
U32 = jnp.uint32
I32 = jnp.int32


def _pack_rne(lo, hi):
    ulo = lax.bitcast_convert_type(lo, U32)
    uhi = lax.bitcast_convert_type(hi, U32)
    ulo = ulo + U32(0x7FFF) + ((ulo >> 16) & U32(1))
    uhi = uhi + U32(0x7FFF) + ((uhi >> 16) & U32(1))
    word = (ulo >> 16) | (uhi & U32(0xFFFF0000))
    return lax.bitcast_convert_type(word, I32)


def _unpack_lo(w):
    u = lax.bitcast_convert_type(w, U32)
    return lax.bitcast_convert_type(u << 16, F32)


def _unpack_hi(w):
    u = lax.bitcast_convert_type(w, U32)
    return lax.bitcast_convert_type(u & U32(0xFFFF0000), F32)

_N = 10000
_E = 320000
_DA = 10
_DF = 128
_DEF = 8
_NP = 10240
_NW = 32
_EPW = _E // _NW
_CH = 80
_NCH = _EPW // _CH
_RPT = _NP // 16


def _node_tables_body(na_ref, nf_ref, wsf_ref, wdf_ref, wup_ref, wres_ref,
                      wskip_ref, tsu_ref, td_ref, res_ref, sc_ref):
    na = na_ref[...]
    nf = nf_ref[...]
    ab = jnp.dot(na, wsf_ref[...], preferred_element_type=F32)
    up = jnp.dot(nf, wup_ref[...], preferred_element_type=F32)
    tsu_ref[...] = _pack_rne(ab, up)
    tdf = jnp.dot(na, wdf_ref[...], preferred_element_type=F32)
    td_ref[:, 0:64] = _pack_rne(tdf[:, 0:64], tdf[:, 64:128])
    td_ref[:, 64:128] = jnp.zeros((na.shape[0], 64), I32)
    res_ref[...] = jnp.dot(nf, wres_ref[...], preferred_element_type=F32)
    sc_ref[...] = jnp.dot(nf, wskip_ref[...], preferred_element_type=F32)


def _node_tables(na, nf, wsf, wdf, wupn, wresn, wskipn):
    tn = 1000
    grid = (_N // tn,)
    full = lambda shape: pl.BlockSpec(shape, lambda i: (0, 0))
    return pl.pallas_call(
        _node_tables_body,
        grid=grid,
        in_specs=[
            pl.BlockSpec((tn, _DA), lambda i: (i, 0)),
            pl.BlockSpec((tn, _DF), lambda i: (i, 0)),
            full((_DA, _DF)), full((_DA, _DF)),
            full((_DF, _DF)), full((_DF, _DF)), full((_DF, _DF)),
        ],
        out_specs=[
            pl.BlockSpec((tn, _DF), lambda i: (i, 0)),
            pl.BlockSpec((tn, _DF), lambda i: (i, 0)),
            pl.BlockSpec((tn, _DF), lambda i: (i, 0)),
            pl.BlockSpec((tn, _DF), lambda i: (i, 0)),
        ],
        out_shape=[
            jax.ShapeDtypeStruct((_N, _DF), I32),
            jax.ShapeDtypeStruct((_N, _DF), I32),
            jax.ShapeDtypeStruct((_N, _DF), F32),
            jax.ShapeDtypeStruct((_N, _DF), F32),
        ],
        compiler_params=pltpu.CompilerParams(
            dimension_semantics=("parallel",)),
    )(na, nf, wsf, wdf, wupn, wresn, wskipn)


def _sc_gather_body(tsu_hbm, td_hbm, srcr_hbm, dstr_hbm, ga_hbm, gb_hbm,
                    sidx, didx, bufs, bufd, sem):
    cid = lax.axis_index("c")
    sid = lax.axis_index("s")
    wid = sid * 2 + cid
    base = wid * _EPW
    pltpu.sync_copy(srcr_hbm.at[wid], sidx)
    pltpu.sync_copy(dstr_hbm.at[wid], didx)

    def start_gather(j, p):
        pltpu.async_copy(tsu_hbm.at[sidx.at[j]], bufs.at[p], sem.at[p])
        pltpu.async_copy(td_hbm.at[didx.at[j]], bufd.at[p], sem.at[p])

    start_gather(0, 0)

    def chunk(j, carry):
        p = lax.rem(j, 2)

        @pl.when(j + 1 < _NCH)
        def _():
            start_gather(j + 1, 1 - p)

        pltpu.make_async_copy(tsu_hbm.at[sidx.at[j]], bufs.at[p],
                              sem.at[p]).wait()
        pltpu.make_async_copy(td_hbm.at[didx.at[j]], bufd.at[p],
                              sem.at[p]).wait()
        b = base + j * _CH
        pltpu.sync_copy(bufs.at[p], ga_hbm.at[pl.ds(b, _CH)])
        pltpu.sync_copy(bufd.at[p], gb_hbm.at[pl.ds(b, _CH)])
        return carry

    lax.fori_loop(0, _NCH, chunk, 0)


def _gather_phase(tsu, td, src, dst):
    sc_gather = pl.kernel(
        _sc_gather_body,
        out_type=(
            jax.ShapeDtypeStruct((_E, _DF), I32),
            jax.ShapeDtypeStruct((_E, _DF), I32),
        ),
        mesh=plsc.VectorSubcoreMesh(core_axis_name="c", subcore_axis_name="s",
                                    num_cores=2, num_subcores=16),
        scratch_types=[
            pltpu.VMEM((_NCH, _CH), jnp.int32),
            pltpu.VMEM((_NCH, _CH), jnp.int32),
            pltpu.VMEM((2, _CH, _DF), I32),
            pltpu.VMEM((2, _CH, _DF), I32),
            pltpu.SemaphoreType.DMA((2,)),
        ],
    )
    return sc_gather(tsu, td, src.reshape(_NW, _NCH, _CH),
                     dst.reshape(_NW, _NCH, _CH))


def _edge_mlp_body(ga_ref, gb_ref, ef_ref, ea_ref, w1e_ref, wde_ref, w2_ref,
                   w3_ref, w4_ref, wd2_ref, mji_ref, dens_ref):
    ga32 = ga_ref[...]
    gb32 = gb_ref[...]
    ab = _unpack_lo(ga32)
    upv = _unpack_hi(ga32)
    gbl = gb32[:, 0:64]
    at = _unpack_lo(gbl)
    bt = _unpack_hi(gbl)
    ef = ef_ref[...]
    h1 = ab[:, 0:64] + at + jnp.dot(ef, w1e_ref[...],
                                    preferred_element_type=F32)
    h1 = h1 * jax.nn.sigmoid(h1)
    d1 = ab[:, 64:128] + bt + jnp.dot(ef, wde_ref[...],
                                      preferred_element_type=F32)
    d1 = d1 * jax.nn.sigmoid(d1)
    h2 = jnp.dot(h1, w2_ref[...], preferred_element_type=F32)
    h2 = h2 * jax.nn.sigmoid(h2)
    h3 = jnp.dot(h2, w3_ref[...], preferred_element_type=F32)
    h3 = h3 * jax.nn.sigmoid(h3)
    tpw = jnp.dot(h3, w4_ref[...], preferred_element_type=F32)
    mji_ref[...] = upv * (ea_ref[...] * tpw)
    dd = jnp.dot(d1, wd2_ref[...], preferred_element_type=F32)
    dens_ref[...] = jnp.tanh(dd * dd)


def _edge_mlp(ga, gb, ef, ea, w1e, wde, w2n, w3n, w4n, wd2n):
    te = 2000
    grid = (_E // te,)
    full = lambda shape: pl.BlockSpec(shape, lambda i: (0, 0))
    return pl.pallas_call(
        _edge_mlp_body,
        grid=grid,
        in_specs=[
            pl.BlockSpec((te, _DF), lambda i: (i, 0)),
            pl.BlockSpec((te, _DF), lambda i: (i, 0)),
            pl.BlockSpec((te, _DEF), lambda i: (i, 0)),
            pl.BlockSpec((te, 1), lambda i: (i, 0)),
            full((_DEF, 64)), full((_DEF, 64)),
            full((64, 64)), full((64, 64)), full((64, _DF)), full((64, 1)),
        ],
        out_specs=[
            pl.BlockSpec((te, _DF), lambda i: (i, 0)),
            pl.BlockSpec((te, 1), lambda i: (i, 0)),
        ],
        out_shape=[
            jax.ShapeDtypeStruct((_E, _DF), F32),
            jax.ShapeDtypeStruct((_E, 1), F32),
        ],
        compiler_params=pltpu.CompilerParams(
            dimension_semantics=("parallel",)),
    )(ga, gb, ef, ea, w1e, wde, w2n, w3n, w4n, wd2n)


def _sc_scatter_body(dstr_hbm, mji_hbm, de_hbm, zm_hbm, zd_hbm,
                     msgp_hbm, denp_hbm,
                     didx, bufm, bufe, sem, acc_m, acc_d):
    cid = lax.axis_index("c")
    sid = lax.axis_index("s")
    wid = sid * 2 + cid
    r0 = sid * _RPT
    pltpu.sync_copy(zm_hbm.at[pl.ds(r0, _RPT)], acc_m.at[pl.ds(r0, _RPT)])
    pltpu.sync_copy(zd_hbm.at[pl.ds(r0, _RPT)], acc_d.at[pl.ds(r0, _RPT)])
    pltpu.sync_copy(dstr_hbm.at[wid], didx)
    plsc.subcore_barrier()
    base = wid * _EPW

    def start_load(j, p):
        b = base + j * _CH
        pltpu.async_copy(mji_hbm.at[pl.ds(b, _CH)], bufm.at[p], sem.at[p])
        pltpu.async_copy(de_hbm.at[pl.ds(b, _CH)], bufe.at[p], sem.at[p])

    start_load(0, 0)

    def chunk(j, carry):
        p = lax.rem(j, 2)

        @pl.when(j + 1 < _NCH)
        def _():
            start_load(j + 1, 1 - p)

        b = base + j * _CH
        pltpu.make_async_copy(mji_hbm.at[pl.ds(b, _CH)], bufm.at[p],
                              sem.at[p]).wait()
        pltpu.make_async_copy(de_hbm.at[pl.ds(b, _CH)], bufe.at[p],
                              sem.at[p]).wait()
        pltpu.sync_copy(bufm.at[p], acc_m.at[didx.at[j]], add=True)
        pltpu.sync_copy(bufe.at[p], acc_d.at[didx.at[j]], add=True)
        return carry

    lax.fori_loop(0, _NCH, chunk, 0)
    plsc.subcore_barrier()
    pltpu.sync_copy(acc_m.at[pl.ds(r0, _RPT)],
                    msgp_hbm.at[cid, pl.ds(r0, _RPT)])
    pltpu.sync_copy(acc_d.at[pl.ds(r0, _RPT)],
                    denp_hbm.at[cid, pl.ds(r0, _RPT)])


def _scatter_phase(dst, mji, de):
    sc_scatter = pl.kernel(
        _sc_scatter_body,
        out_type=(
            jax.ShapeDtypeStruct((2, _NP, _DF), F32),
            jax.ShapeDtypeStruct((2, _NP), F32),
        ),
        mesh=plsc.VectorSubcoreMesh(core_axis_name="c", subcore_axis_name="s",
                                    num_cores=2, num_subcores=16),
        scratch_types=[
            pltpu.VMEM((_NCH, _CH), jnp.int32),
            pltpu.VMEM((2, _CH, _DF), F32),
            pltpu.VMEM((2, _CH), F32),
            pltpu.SemaphoreType.DMA((2,)),
            pltpu.VMEM_SHARED((_NP, _DF), F32),
            pltpu.VMEM_SHARED((_NP,), F32),
        ],
    )
    zm = jnp.zeros((_NP, _DF), F32)
    zd = jnp.zeros((_NP,), F32)
    return sc_scatter(dst.reshape(_NW, _NCH, _CH), mji, de, zm, zd)


def _final_body(msgp_ref, denp_ref, res_ref, w1_ref, w2_ref, a_ref, b_ref,
                out_ref):
    msg = msgp_ref[0] + msgp_ref[1]
    den = denp_ref[0] + denp_ref[1]
    lin = jnp.dot(msg, w1_ref[...], preferred_element_type=F32)
    m = lin / (den * b_ref[0, 0] + a_ref[0, 0]) + res_ref[...]
    m = m * jax.nn.sigmoid(m)
    out_ref[...] = jnp.dot(m, w2_ref[...], preferred_element_type=F32)


def _final(msgp, denp3, resp, w1n, w2n, a2, b2):
    tn = 1024
    grid = (_NP // tn,)
    full = lambda shape: pl.BlockSpec(shape, lambda i: (0, 0))
    smem = pl.BlockSpec((1, 1), lambda i: (0, 0), memory_space=pltpu.SMEM)
    return pl.pallas_call(
        _final_body,
        grid=grid,
        in_specs=[
            pl.BlockSpec((2, tn, _DF), lambda i: (0, i, 0)),
            pl.BlockSpec((2, tn, 1), lambda i: (0, i, 0)),
            pl.BlockSpec((tn, _DF), lambda i: (i, 0)),
            full((_DF, _DF)), full((_DF, _DF)),
            smem, smem,
        ],
        out_specs=pl.BlockSpec((tn, _DF), lambda i: (i, 0)),
        out_shape=jax.ShapeDtypeStruct((_NP, _DF), F32),
        compiler_params=pltpu.CompilerParams(
            dimension_semantics=("parallel",)),
    )(msgp, denp3, resp, w1n, w2n, a2, b2)


def kernel(node_attrs, node_feats, edge_attrs, edge_feats, edge_index,
           W_src, W_tgt, W_up, W_res, W_skip,
           W_tp1, W_tp2, W_tp3, W_tp4, W_d1, W_d2, W_1, W_2, alpha, beta):
    s_attr = math.sqrt(W_src.shape[0])
    s_aug = math.sqrt(W_tp1.shape[0])
    s_mid = math.sqrt(W_tp2.shape[0])
    s_feat = math.sqrt(W_up.shape[0])

    src = edge_index[:, 0].astype(jnp.int32)
    dst = edge_index[:, 1].astype(jnp.int32)

    cfold = 1.0 / (s_attr * s_aug)
    lo, hi = _DEF, _DEF + _DF
    wsf = jnp.concatenate([W_src @ W_tp1[lo:hi], W_src @ W_d1[lo:hi]],
                          axis=1) * cfold
    wdf = jnp.concatenate([W_tgt @ W_tp1[hi:], W_tgt @ W_d1[hi:]],
                          axis=1) * cfold

    tsu, td, resv, scv = _node_tables(
        node_attrs, node_feats, wsf, wdf,
        W_up / s_feat, W_res / s_feat, W_skip / s_feat)

    ga, gb = _gather_phase(tsu, td, src, dst)

    mji, dens_e = _edge_mlp(
        ga, gb, edge_feats, edge_attrs,
        W_tp1[0:_DEF] / s_aug, W_d1[0:_DEF] / s_aug,
        W_tp2 / s_mid, W_tp3 / s_mid, W_tp4 / s_mid, W_d2 / s_mid)

    msgp, denp = _scatter_phase(dst, mji, dens_e.reshape(_E))

    resp = jnp.pad(resv, ((0, _NP - _N), (0, 0)))
    out_m = _final(msgp, denp.reshape(2, _NP, 1), resp,
                   W_1 / s_feat, W_2 / s_feat,
                   alpha.reshape(1, 1), beta.reshape(1, 1))

    return (out_m[:_N].reshape(_N, _DF, 1), scv)


# speedup vs baseline: 4.0948x; 1.0066x over previous
import math

import jax
import jax.numpy as jnp
from jax import lax
from jax.experimental import pallas as pl
from jax.experimental.pallas import tpu as pltpu
from jax.experimental.pallas import tpu_sc as plsc

F32 = jnp.float32
BF16 = jnp.bfloat16
U32 = jnp.uint32
I32 = jnp.int32


def _pack_rne(lo, hi):
    ulo = lax.bitcast_convert_type(lo, U32)
    uhi = lax.bitcast_convert_type(hi, U32)
    ulo = ulo + U32(0x7FFF) + ((ulo >> 16) & U32(1))
    uhi = uhi + U32(0x7FFF) + ((uhi >> 16) & U32(1))
    word = (ulo >> 16) | (uhi & U32(0xFFFF0000))
    return lax.bitcast_convert_type(word, I32)


def _unpack_lo(w):
    u = lax.bitcast_convert_type(w, U32)
    return lax.bitcast_convert_type(u << 16, F32)


def _unpack_hi(w):
    u = lax.bitcast_convert_type(w, U32)
    return lax.bitcast_convert_type(u & U32(0xFFFF0000), F32)

_N = 10000
_E = 320000
_DA = 10
_DF = 128
_DEF = 8
_NP = 10240
_NW = 32
_PIECES = 2
_EH = _E // _PIECES
_EPW = _EH // _NW
_CH = 40
_NCH = _EPW // _CH
_RPT = _NP // 16


def _node_tables_body(na_ref, nf_ref, wsf_ref, wdf_ref, wup_ref, wres_ref,
                      wskip_ref, tsu_ref, td_ref, res_ref, sc_ref):
    na = na_ref[...]
    nf = nf_ref[...]
    ab = jnp.dot(na, wsf_ref[...], preferred_element_type=F32)
    up = jnp.dot(nf, wup_ref[...], preferred_element_type=F32)
    tsu_ref[...] = _pack_rne(ab, up)
    tdf = jnp.dot(na, wdf_ref[...], preferred_element_type=F32)
    td_ref[:, 0:64] = _pack_rne(tdf[:, 0:64], tdf[:, 64:128])
    td_ref[:, 64:128] = jnp.zeros((na.shape[0], 64), I32)
    res_ref[...] = jnp.dot(nf, wres_ref[...], preferred_element_type=F32)
    sc_ref[...] = jnp.dot(nf, wskip_ref[...], preferred_element_type=F32)


def _node_tables(na, nf, wsf, wdf, wupn, wresn, wskipn):
    tn = 1000
    grid = (_N // tn,)
    full = lambda shape: pl.BlockSpec(shape, lambda i: (0, 0))
    return pl.pallas_call(
        _node_tables_body,
        grid=grid,
        in_specs=[
            pl.BlockSpec((tn, _DA), lambda i: (i, 0)),
            pl.BlockSpec((tn, _DF), lambda i: (i, 0)),
            full((_DA, _DF)), full((_DA, _DF)),
            full((_DF, _DF)), full((_DF, _DF)), full((_DF, _DF)),
        ],
        out_specs=[
            pl.BlockSpec((tn, _DF), lambda i: (i, 0)),
            pl.BlockSpec((tn, _DF), lambda i: (i, 0)),
            pl.BlockSpec((tn, _DF), lambda i: (i, 0)),
            pl.BlockSpec((tn, _DF), lambda i: (i, 0)),
        ],
        out_shape=[
            jax.ShapeDtypeStruct((_N, _DF), I32),
            jax.ShapeDtypeStruct((_N, _DF), I32),
            jax.ShapeDtypeStruct((_N, _DF), F32),
            jax.ShapeDtypeStruct((_N, _DF), F32),
        ],
        compiler_params=pltpu.CompilerParams(
            dimension_semantics=("parallel",)),
    )(na, nf, wsf, wdf, wupn, wresn, wskipn)


def _sc_gather_body(tsu_hbm, td_hbm, srcr_hbm, dstr_hbm, ga_hbm, gb_hbm,
                    sidx, didx, bufs, bufd, sem):
    cid = lax.axis_index("c")
    sid = lax.axis_index("s")
    wid = sid * 2 + cid
    base = wid * _EPW
    pltpu.sync_copy(srcr_hbm.at[wid], sidx)
    pltpu.sync_copy(dstr_hbm.at[wid], didx)

    def start_gather(j, p):
        pltpu.async_copy(tsu_hbm.at[sidx.at[j]], bufs.at[p], sem.at[p])
        pltpu.async_copy(td_hbm.at[didx.at[j]], bufd.at[p], sem.at[p])

    start_gather(0, 0)

    def chunk(j, carry):
        p = lax.rem(j, 2)

        @pl.when(j + 1 < _NCH)
        def _():
            start_gather(j + 1, 1 - p)

        pltpu.make_async_copy(tsu_hbm.at[sidx.at[j]], bufs.at[p],
                              sem.at[p]).wait()
        pltpu.make_async_copy(td_hbm.at[didx.at[j]], bufd.at[p],
                              sem.at[p]).wait()
        b = base + j * _CH
        pltpu.sync_copy(bufs.at[p], ga_hbm.at[pl.ds(b, _CH)])
        pltpu.sync_copy(bufd.at[p], gb_hbm.at[pl.ds(b, _CH)])
        return carry

    lax.fori_loop(0, _NCH, chunk, 0)


def _gather_phase(tsu, td, src, dst):
    sc_gather = pl.kernel(
        _sc_gather_body,
        out_type=(
            jax.ShapeDtypeStruct((_EH, _DF), I32),
            jax.ShapeDtypeStruct((_EH, _DF), I32),
        ),
        mesh=plsc.VectorSubcoreMesh(core_axis_name="c", subcore_axis_name="s",
                                    num_cores=2, num_subcores=16),
        scratch_types=[
            pltpu.VMEM((_NCH, _CH), jnp.int32),
            pltpu.VMEM((_NCH, _CH), jnp.int32),
            pltpu.VMEM((2, _CH, _DF), I32),
            pltpu.VMEM((2, _CH, _DF), I32),
            pltpu.SemaphoreType.DMA((2,)),
        ],
    )
    return sc_gather(tsu, td, src.reshape(_NW, _NCH, _CH),
                     dst.reshape(_NW, _NCH, _CH))


def _edge_mlp_body(ga_ref, gb_ref, ef_ref, ea_ref, w1e_ref, wde_ref, w2_ref,
                   w3_ref, w4_ref, wd2_ref, mji_ref, dens_ref):
    ga32 = ga_ref[...]
    gb32 = gb_ref[...]
    ab = _unpack_lo(ga32)
    upv = _unpack_hi(ga32)
    gbl = gb32[:, 0:64]
    at = _unpack_lo(gbl)
    bt = _unpack_hi(gbl)
    ef = ef_ref[...]
    h1 = ab[:, 0:64] + at + jnp.dot(ef, w1e_ref[...],
                                    preferred_element_type=F32)
    h1 = h1 * jax.nn.sigmoid(h1)
    d1 = ab[:, 64:128] + bt + jnp.dot(ef, wde_ref[...],
                                      preferred_element_type=F32)
    d1 = d1 * jax.nn.sigmoid(d1)
    h2 = jnp.dot(h1, w2_ref[...], preferred_element_type=F32)
    h2 = h2 * jax.nn.sigmoid(h2)
    h3 = jnp.dot(h2, w3_ref[...], preferred_element_type=F32)
    h3 = h3 * jax.nn.sigmoid(h3)
    tpw = jnp.dot(h3, w4_ref[...], preferred_element_type=F32)
    mji_ref[...] = upv * (ea_ref[...] * tpw)
    dd = jnp.dot(d1, wd2_ref[...], preferred_element_type=F32)
    dens_ref[...] = jnp.tanh(dd * dd)


def _edge_mlp(ga, gb, ef, ea, w1e, wde, w2n, w3n, w4n, wd2n):
    te = 2000
    grid = (_EH // te,)
    full = lambda shape: pl.BlockSpec(shape, lambda i: (0, 0))
    return pl.pallas_call(
        _edge_mlp_body,
        grid=grid,
        in_specs=[
            pl.BlockSpec((te, _DF), lambda i: (i, 0)),
            pl.BlockSpec((te, _DF), lambda i: (i, 0)),
            pl.BlockSpec((te, _DEF), lambda i: (i, 0)),
            pl.BlockSpec((te, 1), lambda i: (i, 0)),
            full((_DEF, 64)), full((_DEF, 64)),
            full((64, 64)), full((64, 64)), full((64, _DF)), full((64, 1)),
        ],
        out_specs=[
            pl.BlockSpec((te, _DF), lambda i: (i, 0)),
            pl.BlockSpec((te, 1), lambda i: (i, 0)),
        ],
        out_shape=[
            jax.ShapeDtypeStruct((_EH, _DF), F32),
            jax.ShapeDtypeStruct((_EH, 1), F32),
        ],
        compiler_params=pltpu.CompilerParams(
            dimension_semantics=("parallel",)),
    )(ga, gb, ef, ea, w1e, wde, w2n, w3n, w4n, wd2n)


def _sc_scatter_body(dstr_hbm, mji_hbm, de_hbm, zm_hbm, zd_hbm,
                     msgp_hbm, denp_hbm,
                     didx, bufm, bufe, sem, acc_m, acc_d):
    cid = lax.axis_index("c")
    sid = lax.axis_index("s")
    wid = sid * 2 + cid
    r0 = sid * _RPT
    pltpu.sync_copy(zm_hbm.at[cid, pl.ds(r0, _RPT)],
                    acc_m.at[pl.ds(r0, _RPT)])
    pltpu.sync_copy(zd_hbm.at[cid, pl.ds(r0, _RPT)],
                    acc_d.at[pl.ds(r0, _RPT)])
    pltpu.sync_copy(dstr_hbm.at[wid], didx)
    plsc.subcore_barrier()
    base = wid * _EPW

    def start_load(j, p):
        b = base + j * _CH
        pltpu.async_copy(mji_hbm.at[pl.ds(b, _CH)], bufm.at[p], sem.at[p])
        pltpu.async_copy(de_hbm.at[pl.ds(b, _CH)], bufe.at[p], sem.at[p])

    start_load(0, 0)

    def chunk(j, carry):
        p = lax.rem(j, 2)

        @pl.when(j + 1 < _NCH)
        def _():
            start_load(j + 1, 1 - p)

        b = base + j * _CH
        pltpu.make_async_copy(mji_hbm.at[pl.ds(b, _CH)], bufm.at[p],
                              sem.at[p]).wait()
        pltpu.make_async_copy(de_hbm.at[pl.ds(b, _CH)], bufe.at[p],
                              sem.at[p]).wait()
        pltpu.sync_copy(bufm.at[p], acc_m.at[didx.at[j]], add=True)
        pltpu.sync_copy(bufe.at[p], acc_d.at[didx.at[j]], add=True)
        return carry

    lax.fori_loop(0, _NCH, chunk, 0)
    plsc.subcore_barrier()
    pltpu.sync_copy(acc_m.at[pl.ds(r0, _RPT)],
                    msgp_hbm.at[cid, pl.ds(r0, _RPT)])
    pltpu.sync_copy(acc_d.at[pl.ds(r0, _RPT)],
                    denp_hbm.at[cid, pl.ds(r0, _RPT)])


def _scatter_phase(dst, mji, de, init_m, init_d):
    sc_scatter = pl.kernel(
        _sc_scatter_body,
        out_type=(
            jax.ShapeDtypeStruct((2, _NP, _DF), F32),
            jax.ShapeDtypeStruct((2, _NP), F32),
        ),
        mesh=plsc.VectorSubcoreMesh(core_axis_name="c", subcore_axis_name="s",
                                    num_cores=2, num_subcores=16),
        scratch_types=[
            pltpu.VMEM((_NCH, _CH), jnp.int32),
            pltpu.VMEM((2, _CH, _DF), F32),
            pltpu.VMEM((2, _CH), F32),
            pltpu.SemaphoreType.DMA((2,)),
            pltpu.VMEM_SHARED((_NP, _DF), F32),
            pltpu.VMEM_SHARED((_NP,), F32),
        ],
    )
    return sc_scatter(dst.reshape(_NW, _NCH, _CH), mji, de, init_m, init_d)


def _final_body(msgp_ref, denp_ref, res_ref, w1_ref, w2_ref, a_ref, b_ref,
                out_ref):
    msg = msgp_ref[0] + msgp_ref[1]
    den = denp_ref[0] + denp_ref[1]
    lin = jnp.dot(msg, w1_ref[...], preferred_element_type=F32)
    m = lin / (den * b_ref[0, 0] + a_ref[0, 0]) + res_ref[...]
    m = m * jax.nn.sigmoid(m)
    out_ref[...] = jnp.dot(m, w2_ref[...], preferred_element_type=F32)


def _final(msgp, denp3, resp, w1n, w2n, a2, b2):
    tn = 1024
    grid = (_NP // tn,)
    full = lambda shape: pl.BlockSpec(shape, lambda i: (0, 0))
    smem = pl.BlockSpec((1, 1), lambda i: (0, 0), memory_space=pltpu.SMEM)
    return pl.pallas_call(
        _final_body,
        grid=grid,
        in_specs=[
            pl.BlockSpec((2, tn, _DF), lambda i: (0, i, 0)),
            pl.BlockSpec((2, tn, 1), lambda i: (0, i, 0)),
            pl.BlockSpec((tn, _DF), lambda i: (i, 0)),
            full((_DF, _DF)), full((_DF, _DF)),
            smem, smem,
        ],
        out_specs=pl.BlockSpec((tn, _DF), lambda i: (i, 0)),
        out_shape=jax.ShapeDtypeStruct((_NP, _DF), F32),
        compiler_params=pltpu.CompilerParams(
            dimension_semantics=("parallel",)),
    )(msgp, denp3, resp, w1n, w2n, a2, b2)


def kernel(node_attrs, node_feats, edge_attrs, edge_feats, edge_index,
           W_src, W_tgt, W_up, W_res, W_skip,
           W_tp1, W_tp2, W_tp3, W_tp4, W_d1, W_d2, W_1, W_2, alpha, beta):
    s_attr = math.sqrt(W_src.shape[0])
    s_aug = math.sqrt(W_tp1.shape[0])
    s_mid = math.sqrt(W_tp2.shape[0])
    s_feat = math.sqrt(W_up.shape[0])

    src = edge_index[:, 0].astype(jnp.int32)
    dst = edge_index[:, 1].astype(jnp.int32)

    cfold = 1.0 / (s_attr * s_aug)
    lo, hi = _DEF, _DEF + _DF
    wsf = jnp.concatenate([W_src @ W_tp1[lo:hi], W_src @ W_d1[lo:hi]],
                          axis=1) * cfold
    wdf = jnp.concatenate([W_tgt @ W_tp1[hi:], W_tgt @ W_d1[hi:]],
                          axis=1) * cfold

    tsu, td, resv, scv = _node_tables(
        node_attrs, node_feats, wsf, wdf,
        W_up / s_feat, W_res / s_feat, W_skip / s_feat)

    msgp = jnp.zeros((2, _NP, _DF), F32)
    denp = jnp.zeros((2, _NP), F32)
    for i in range(_PIECES):
        lo, hi2 = i * _EH, (i + 1) * _EH
        ga, gb = _gather_phase(tsu, td, src[lo:hi2], dst[lo:hi2])
        mji, dens_e = _edge_mlp(
            ga, gb, edge_feats[lo:hi2], edge_attrs[lo:hi2],
            W_tp1[0:_DEF] / s_aug, W_d1[0:_DEF] / s_aug,
            W_tp2 / s_mid, W_tp3 / s_mid, W_tp4 / s_mid, W_d2 / s_mid)
        msgp, denp = _scatter_phase(dst[lo:hi2], mji, dens_e.reshape(_EH),
                                    msgp, denp)

    resp = jnp.pad(resv, ((0, _NP - _N), (0, 0)))
    out_m = _final(msgp, denp.reshape(2, _NP, 1), resp,
                   W_1 / s_feat, W_2 / s_feat,
                   alpha.reshape(1, 1), beta.reshape(1, 1))

    return (out_m[:_N].reshape(_N, _DF, 1), scv)


# speedup vs baseline: 4.2328x; 1.0337x over previous
import math

import jax
import jax.numpy as jnp
from jax import lax
from jax.experimental import pallas as pl
from jax.experimental.pallas import tpu as pltpu
from jax.experimental.pallas import tpu_sc as plsc

F32 = jnp.float32
BF16 = jnp.bfloat16
U32 = jnp.uint32
I32 = jnp.int32


def _pack_rne(lo, hi):
    ulo = lax.bitcast_convert_type(lo, U32)
    uhi = lax.bitcast_convert_type(hi, U32)
    ulo = ulo + U32(0x7FFF) + ((ulo >> 16) & U32(1))
    uhi = uhi + U32(0x7FFF) + ((uhi >> 16) & U32(1))
    word = (ulo >> 16) | (uhi & U32(0xFFFF0000))
    return lax.bitcast_convert_type(word, I32)


def _unpack_lo(w):
    u = lax.bitcast_convert_type(w, U32)
    return lax.bitcast_convert_type(u << 16, F32)


def _unpack_hi(w):
    u = lax.bitcast_convert_type(w, U32)
    return lax.bitcast_convert_type(u & U32(0xFFFF0000), F32)

_N = 10000
_E = 320000
_DA = 10
_DF = 128
_DEF = 8
_NP = 10240
_NW = 32
_PIECES = 1
_EH = _E // _PIECES
_EPW = _EH // _NW
_CH = 80
_NCH = _EPW // _CH
_RPT = _NP // 16


def _node_tables_body(na_ref, nf_ref, wsf_ref, wdf_ref, wup_ref, wres_ref,
                      wskip_ref, tsu_ref, td_ref, res_ref, sc_ref):
    na = na_ref[...]
    nf = nf_ref[...]
    ab = jnp.dot(na, wsf_ref[...], preferred_element_type=F32)
    up = jnp.dot(nf, wup_ref[...], preferred_element_type=F32)
    tsu_ref[...] = _pack_rne(ab, up)
    tdf = jnp.dot(na, wdf_ref[...], preferred_element_type=F32)
    td_ref[:, 0:64] = _pack_rne(tdf[:, 0:64], tdf[:, 64:128])
    td_ref[:, 64:128] = jnp.zeros((na.shape[0], 64), I32)
    res_ref[...] = jnp.dot(nf, wres_ref[...], preferred_element_type=F32)
    sc_ref[...] = jnp.dot(nf, wskip_ref[...], preferred_element_type=F32)


def _node_tables(na, nf, wsf, wdf, wupn, wresn, wskipn):
    tn = 1000
    grid = (_N // tn,)
    full = lambda shape: pl.BlockSpec(shape, lambda i: (0, 0))
    return pl.pallas_call(
        _node_tables_body,
        grid=grid,
        in_specs=[
            pl.BlockSpec((tn, _DA), lambda i: (i, 0)),
            pl.BlockSpec((tn, _DF), lambda i: (i, 0)),
            full((_DA, _DF)), full((_DA, _DF)),
            full((_DF, _DF)), full((_DF, _DF)), full((_DF, _DF)),
        ],
        out_specs=[
            pl.BlockSpec((tn, _DF), lambda i: (i, 0)),
            pl.BlockSpec((tn, _DF), lambda i: (i, 0)),
            pl.BlockSpec((tn, _DF), lambda i: (i, 0)),
            pl.BlockSpec((tn, _DF), lambda i: (i, 0)),
        ],
        out_shape=[
            jax.ShapeDtypeStruct((_N, _DF), I32),
            jax.ShapeDtypeStruct((_N, _DF), I32),
            jax.ShapeDtypeStruct((_N, _DF), F32),
            jax.ShapeDtypeStruct((_N, _DF), F32),
        ],
        compiler_params=pltpu.CompilerParams(
            dimension_semantics=("parallel",)),
    )(na, nf, wsf, wdf, wupn, wresn, wskipn)


def _sc_gather_body(tsu_hbm, td_hbm, srcr_hbm, dstr_hbm, ga_hbm, gb_hbm,
                    sidx, didx, bufs, bufd, sem, wsem):
    cid = lax.axis_index("c")
    sid = lax.axis_index("s")
    wid = sid * 2 + cid
    base = wid * _EPW
    pltpu.sync_copy(srcr_hbm.at[wid], sidx)
    pltpu.sync_copy(dstr_hbm.at[wid], didx)

    def start_gather(j, p):
        pltpu.async_copy(tsu_hbm.at[sidx.at[j]], bufs.at[p], sem.at[p])
        pltpu.async_copy(td_hbm.at[didx.at[j]], bufd.at[p], sem.at[p])

    def wait_write(j, p):
        b = base + j * _CH
        pltpu.make_async_copy(bufs.at[p], ga_hbm.at[pl.ds(b, _CH)],
                              wsem.at[p]).wait()
        pltpu.make_async_copy(bufd.at[p], gb_hbm.at[pl.ds(b, _CH)],
                              wsem.at[p]).wait()

    start_gather(0, 0)

    def chunk(j, carry):
        p = lax.rem(j, 2)

        @pl.when(j + 1 < _NCH)
        def _():
            @pl.when(j >= 1)
            def _():
                wait_write(j - 1, 1 - p)

            start_gather(j + 1, 1 - p)

        pltpu.make_async_copy(tsu_hbm.at[sidx.at[j]], bufs.at[p],
                              sem.at[p]).wait()
        pltpu.make_async_copy(td_hbm.at[didx.at[j]], bufd.at[p],
                              sem.at[p]).wait()
        b = base + j * _CH
        pltpu.async_copy(bufs.at[p], ga_hbm.at[pl.ds(b, _CH)], wsem.at[p])
        pltpu.async_copy(bufd.at[p], gb_hbm.at[pl.ds(b, _CH)], wsem.at[p])
        return carry

    lax.fori_loop(0, _NCH, chunk, 0)
    wait_write(_NCH - 2, (_NCH - 2) % 2)
    wait_write(_NCH - 1, (_NCH - 1) % 2)


def _gather_phase(tsu, td, src, dst):
    sc_gather = pl.kernel(
        _sc_gather_body,
        out_type=(
            jax.ShapeDtypeStruct((_EH, _DF), I32),
            jax.ShapeDtypeStruct((_EH, _DF), I32),
        ),
        mesh=plsc.VectorSubcoreMesh(core_axis_name="c", subcore_axis_name="s",
                                    num_cores=2, num_subcores=16),
        scratch_types=[
            pltpu.VMEM((_NCH, _CH), jnp.int32),
            pltpu.VMEM((_NCH, _CH), jnp.int32),
            pltpu.VMEM((2, _CH, _DF), I32),
            pltpu.VMEM((2, _CH, _DF), I32),
            pltpu.SemaphoreType.DMA((2,)),
            pltpu.SemaphoreType.DMA((2,)),
        ],
    )
    return sc_gather(tsu, td, src.reshape(_NW, _NCH, _CH),
                     dst.reshape(_NW, _NCH, _CH))


def _edge_mlp_body(ga_ref, gb_ref, ef_ref, ea_ref, w1e_ref, wde_ref, w2_ref,
                   w3_ref, w4_ref, wd2_ref, mji_ref, dens_ref):
    ga32 = ga_ref[...]
    gb32 = gb_ref[...]
    ab = _unpack_lo(ga32)
    upv = _unpack_hi(ga32)
    gbl = gb32[:, 0:64]
    at = _unpack_lo(gbl)
    bt = _unpack_hi(gbl)
    ef = ef_ref[...]
    h1 = ab[:, 0:64] + at + jnp.dot(ef, w1e_ref[...],
                                    preferred_element_type=F32)
    h1 = h1 * jax.nn.sigmoid(h1)
    d1 = ab[:, 64:128] + bt + jnp.dot(ef, wde_ref[...],
                                      preferred_element_type=F32)
    d1 = d1 * jax.nn.sigmoid(d1)
    h2 = jnp.dot(h1, w2_ref[...], preferred_element_type=F32)
    h2 = h2 * jax.nn.sigmoid(h2)
    h3 = jnp.dot(h2, w3_ref[...], preferred_element_type=F32)
    h3 = h3 * jax.nn.sigmoid(h3)
    tpw = jnp.dot(h3, w4_ref[...], preferred_element_type=F32)
    mji_ref[...] = upv * (ea_ref[...] * tpw)
    dd = jnp.dot(d1, wd2_ref[...], preferred_element_type=F32)
    dens_ref[...] = jnp.tanh(dd * dd)


def _edge_mlp(ga, gb, ef, ea, w1e, wde, w2n, w3n, w4n, wd2n):
    te = 4000
    grid = (_EH // te,)
    full = lambda shape: pl.BlockSpec(shape, lambda i: (0, 0))
    return pl.pallas_call(
        _edge_mlp_body,
        grid=grid,
        in_specs=[
            pl.BlockSpec((te, _DF), lambda i: (i, 0)),
            pl.BlockSpec((te, _DF), lambda i: (i, 0)),
            pl.BlockSpec((te, _DEF), lambda i: (i, 0)),
            pl.BlockSpec((te, 1), lambda i: (i, 0)),
            full((_DEF, 64)), full((_DEF, 64)),
            full((64, 64)), full((64, 64)), full((64, _DF)), full((64, 1)),
        ],
        out_specs=[
            pl.BlockSpec((te, _DF), lambda i: (i, 0)),
            pl.BlockSpec((te, 1), lambda i: (i, 0)),
        ],
        out_shape=[
            jax.ShapeDtypeStruct((_EH, _DF), F32),
            jax.ShapeDtypeStruct((_EH, 1), F32),
        ],
        compiler_params=pltpu.CompilerParams(
            dimension_semantics=("parallel",)),
    )(ga, gb, ef, ea, w1e, wde, w2n, w3n, w4n, wd2n)


def _sc_scatter_body(dstr_hbm, mji_hbm, de_hbm, zm_hbm, zd_hbm,
                     msgp_hbm, denp_hbm,
                     didx, bufm, bufe, sem, acc_m, acc_d):
    cid = lax.axis_index("c")
    sid = lax.axis_index("s")
    wid = sid * 2 + cid
    r0 = sid * _RPT
    pltpu.sync_copy(zm_hbm.at[cid, pl.ds(r0, _RPT)],
                    acc_m.at[pl.ds(r0, _RPT)])
    pltpu.sync_copy(zd_hbm.at[cid, pl.ds(r0, _RPT)],
                    acc_d.at[pl.ds(r0, _RPT)])
    pltpu.sync_copy(dstr_hbm.at[wid], didx)
    plsc.subcore_barrier()
    base = wid * _EPW

    def start_load(j, p):
        b = base + j * _CH
        pltpu.async_copy(mji_hbm.at[pl.ds(b, _CH)], bufm.at[p], sem.at[p])
        pltpu.async_copy(de_hbm.at[pl.ds(b, _CH)], bufe.at[p], sem.at[p])

    start_load(0, 0)

    def chunk(j, carry):
        p = lax.rem(j, 2)

        @pl.when(j + 1 < _NCH)
        def _():
            start_load(j + 1, 1 - p)

        b = base + j * _CH
        pltpu.make_async_copy(mji_hbm.at[pl.ds(b, _CH)], bufm.at[p],
                              sem.at[p]).wait()
        pltpu.make_async_copy(de_hbm.at[pl.ds(b, _CH)], bufe.at[p],
                              sem.at[p]).wait()
        pltpu.sync_copy(bufm.at[p], acc_m.at[didx.at[j]], add=True)
        pltpu.sync_copy(bufe.at[p], acc_d.at[didx.at[j]], add=True)
        return carry

    lax.fori_loop(0, _NCH, chunk, 0)
    plsc.subcore_barrier()
    pltpu.sync_copy(acc_m.at[pl.ds(r0, _RPT)],
                    msgp_hbm.at[cid, pl.ds(r0, _RPT)])
    pltpu.sync_copy(acc_d.at[pl.ds(r0, _RPT)],
                    denp_hbm.at[cid, pl.ds(r0, _RPT)])


def _scatter_phase(dst, mji, de, init_m, init_d):
    sc_scatter = pl.kernel(
        _sc_scatter_body,
        out_type=(
            jax.ShapeDtypeStruct((2, _NP, _DF), F32),
            jax.ShapeDtypeStruct((2, _NP), F32),
        ),
        mesh=plsc.VectorSubcoreMesh(core_axis_name="c", subcore_axis_name="s",
                                    num_cores=2, num_subcores=16),
        scratch_types=[
            pltpu.VMEM((_NCH, _CH), jnp.int32),
            pltpu.VMEM((2, _CH, _DF), F32),
            pltpu.VMEM((2, _CH), F32),
            pltpu.SemaphoreType.DMA((2,)),
            pltpu.VMEM_SHARED((_NP, _DF), F32),
            pltpu.VMEM_SHARED((_NP,), F32),
        ],
    )
    return sc_scatter(dst.reshape(_NW, _NCH, _CH), mji, de, init_m, init_d)


def _final_body(msgp_ref, denp_ref, res_ref, w1_ref, w2_ref, a_ref, b_ref,
                out_ref):
    msg = msgp_ref[0] + msgp_ref[1]
    den = denp_ref[0] + denp_ref[1]
    lin = jnp.dot(msg, w1_ref[...], preferred_element_type=F32)
    m = lin / (den * b_ref[0, 0] + a_ref[0, 0]) + res_ref[...]
    m = m * jax.nn.sigmoid(m)
    out_ref[...] = jnp.dot(m, w2_ref[...], preferred_element_type=F32)


def _final(msgp, denp3, resp, w1n, w2n, a2, b2):
    tn = 1024
    grid = (_NP // tn,)
    full = lambda shape: pl.BlockSpec(shape, lambda i: (0, 0))
    smem = pl.BlockSpec((1, 1), lambda i: (0, 0), memory_space=pltpu.SMEM)
    return pl.pallas_call(
        _final_body,
        grid=grid,
        in_specs=[
            pl.BlockSpec((2, tn, _DF), lambda i: (0, i, 0)),
            pl.BlockSpec((2, tn, 1), lambda i: (0, i, 0)),
            pl.BlockSpec((tn, _DF), lambda i: (i, 0)),
            full((_DF, _DF)), full((_DF, _DF)),
            smem, smem,
        ],
        out_specs=pl.BlockSpec((tn, _DF), lambda i: (i, 0)),
        out_shape=jax.ShapeDtypeStruct((_NP, _DF), F32),
        compiler_params=pltpu.CompilerParams(
            dimension_semantics=("parallel",)),
    )(msgp, denp3, resp, w1n, w2n, a2, b2)


def kernel(node_attrs, node_feats, edge_attrs, edge_feats, edge_index,
           W_src, W_tgt, W_up, W_res, W_skip,
           W_tp1, W_tp2, W_tp3, W_tp4, W_d1, W_d2, W_1, W_2, alpha, beta):
    s_attr = math.sqrt(W_src.shape[0])
    s_aug = math.sqrt(W_tp1.shape[0])
    s_mid = math.sqrt(W_tp2.shape[0])
    s_feat = math.sqrt(W_up.shape[0])

    src = edge_index[:, 0].astype(jnp.int32)
    dst = edge_index[:, 1].astype(jnp.int32)

    cfold = 1.0 / (s_attr * s_aug)
    lo, hi = _DEF, _DEF + _DF
    wsf = jnp.concatenate([W_src @ W_tp1[lo:hi], W_src @ W_d1[lo:hi]],
                          axis=1) * cfold
    wdf = jnp.concatenate([W_tgt @ W_tp1[hi:], W_tgt @ W_d1[hi:]],
                          axis=1) * cfold

    tsu, td, resv, scv = _node_tables(
        node_attrs, node_feats, wsf, wdf,
        W_up / s_feat, W_res / s_feat, W_skip / s_feat)

    msgp = jnp.zeros((2, _NP, _DF), F32)
    denp = jnp.zeros((2, _NP), F32)
    for i in range(_PIECES):
        lo, hi2 = i * _EH, (i + 1) * _EH
        ga, gb = _gather_phase(tsu, td, src[lo:hi2], dst[lo:hi2])
        mji, dens_e = _edge_mlp(
            ga, gb, edge_feats[lo:hi2], edge_attrs[lo:hi2],
            W_tp1[0:_DEF] / s_aug, W_d1[0:_DEF] / s_aug,
            W_tp2 / s_mid, W_tp3 / s_mid, W_tp4 / s_mid, W_d2 / s_mid)
        msgp, denp = _scatter_phase(dst[lo:hi2], mji, dens_e.reshape(_EH),
                                    msgp, denp)

    resp = jnp.pad(resv, ((0, _NP - _N), (0, 0)))
    out_m = _final(msgp, denp.reshape(2, _NP, 1), resp,
                   W_1 / s_feat, W_2 / s_feat,
                   alpha.reshape(1, 1), beta.reshape(1, 1))

    return (out_m[:_N].reshape(_N, _DF, 1), scv)
